# Initial kernel scaffold; baseline (speedup 1.0000x reference)
import jax
import jax.numpy as jnp
from jax.experimental import pallas as pl


def kernel(enc, dec, W_pre, b_pre, W_post, b_post, codebook):
    raise NotImplementedError("write your pallas kernel here")


# speedup vs baseline: 2.3286x; 2.3286x over previous
import jax
import jax.numpy as jnp
from jax.experimental import pallas as pl
from jax.experimental.pallas import tpu as pltpu

_B, _T, _C, _K = 16, 2048, 256, 1024
_M = 512
_NJ = _T // _M


def _fused_tc(enc_ref, dec_ref, wpre_ref, bpre_ref, wpost_ref, bpost_ref,
              cbt_ref, cb_ref,
              out_ref, idx_ref, cm_ref, kl_ref,
              cw_ref, c2_ref, cnt_ref):
    b = pl.program_id(0)
    j = pl.program_id(1)

    @pl.when((b == 0) & (j == 0))
    def _init_consts():
        cbt = cbt_ref[...]
        c2_ref[...] = jnp.sum(cbt * cbt, axis=0, keepdims=True)
        cw_ref[...] = jnp.dot(cb_ref[...], wpost_ref[...],
                              preferred_element_type=jnp.float32)

    @pl.when(j == 0)
    def _init_batch():
        cm_ref[...] = jnp.zeros_like(cm_ref)
        cnt_ref[...] = jnp.zeros_like(cnt_ref)

    x = enc_ref[0] - dec_ref[0]
    r = jnp.dot(x, wpre_ref[...],
                preferred_element_type=jnp.float32) + bpre_ref[...]
    dots = jnp.dot(r, cbt_ref[...],
                   preferred_element_type=jnp.float32)
    r2 = jnp.sum(r * r, axis=1, keepdims=True)
    dists = (r2 - 2.0 * dots) + c2_ref[...]

    idx = jnp.argmin(dists, axis=1)
    mind = jnp.min(dists, axis=1)
    idx_ref[0, 0, :] = idx

    lane = jax.lax.broadcasted_iota(jnp.int32, (_M, _K), 1)
    oh = (lane == idx[:, None]).astype(jnp.float32)
    cnt_ref[...] += jnp.sum(oh, axis=0, keepdims=True)
    cm_ref[...] += jnp.sum(mind).reshape(1, 1, 1)

    quant = jnp.dot(oh.astype(jnp.bfloat16), cw_ref[...].astype(jnp.bfloat16),
                    preferred_element_type=jnp.float32)
    y = jnp.dot(dec_ref[0], wpost_ref[...],
                preferred_element_type=jnp.float32)
    out_ref[0] = y + quant + bpost_ref[...]

    @pl.when(j == _NJ - 1)
    def _finalize_batch():
        cm_ref[...] = cm_ref[...] * (1.0 / (_T * _C))
        p = cnt_ref[...] * (1.0 / _T)
        kl_ref[...] = jnp.sum(p * jnp.log(p * _K + 1e-10)).reshape(1, 1, 1)


def kernel(enc, dec, W_pre, b_pre, W_post, b_post, codebook):
    cbt = codebook.T
    bpre2 = b_pre.reshape(1, _C)
    bpost2 = b_post.reshape(1, _C)

    out, idx3, cm3, kl3 = pl.pallas_call(
        _fused_tc,
        grid=(_B, _NJ),
        in_specs=[
            pl.BlockSpec((1, _M, _C), lambda b, j: (b, j, 0)),
            pl.BlockSpec((1, _M, _C), lambda b, j: (b, j, 0)),
            pl.BlockSpec((_C, _C), lambda b, j: (0, 0)),
            pl.BlockSpec((1, _C), lambda b, j: (0, 0)),
            pl.BlockSpec((_C, _C), lambda b, j: (0, 0)),
            pl.BlockSpec((1, _C), lambda b, j: (0, 0)),
            pl.BlockSpec((_C, _K), lambda b, j: (0, 0)),
            pl.BlockSpec((_K, _C), lambda b, j: (0, 0)),
        ],
        out_specs=[
            pl.BlockSpec((1, _M, _C), lambda b, j: (b, j, 0)),
            pl.BlockSpec((1, 1, _M), lambda b, j: (b * _NJ + j, 0, 0)),
            pl.BlockSpec((1, 1, 1), lambda b, j: (b, 0, 0)),
            pl.BlockSpec((1, 1, 1), lambda b, j: (b, 0, 0)),
        ],
        out_shape=[
            jax.ShapeDtypeStruct((_B, _T, _C), jnp.float32),
            jax.ShapeDtypeStruct((_B * _NJ, 1, _M), jnp.int32),
            jax.ShapeDtypeStruct((_B, 1, 1), jnp.float32),
            jax.ShapeDtypeStruct((_B, 1, 1), jnp.float32),
        ],
        scratch_shapes=[
            pltpu.VMEM((_K, _C), jnp.float32),
            pltpu.VMEM((1, _K), jnp.float32),
            pltpu.VMEM((1, _K), jnp.float32),
        ],
    )(enc, dec, W_pre, bpre2, W_post, bpost2, cbt, codebook)

    indices = idx3.reshape(_B, _T)
    cm = cm3.reshape(_B)
    kl = kl3.reshape(_B)
    return out, cm, cm, kl, indices
